# Initial kernel scaffold; baseline (speedup 1.0000x reference)
import jax
import jax.numpy as jnp
from jax.experimental import pallas as pl


def kernel(user_id, movie_id, gender, age, occupation, zip_code, user_emb, movie_emb, gender_emb, age_emb, occupation_emb, zip_emb, W, b):
    raise NotImplementedError("write your pallas kernel here")


# speedup vs baseline: 7.1686x; 7.1686x over previous
import functools

import jax
import jax.numpy as jnp
from jax import lax
from jax.experimental import pallas as pl
from jax.experimental.pallas import tpu as pltpu
from jax.experimental.pallas import tpu_sc as plsc

B = 16384
ED = 128
MD = 8
CT = 136

_info = plsc.get_sparse_core_info()
NC, NS = _info.num_cores, _info.num_subcores
NW = NC * NS
BPW = B // NW
CH = 128
NCH = BPW // CH


def _sc_gather(uid, mid, uemb, memb):
    mesh = plsc.VectorSubcoreMesh(core_axis_name="c", subcore_axis_name="s")

    @functools.partial(
        pl.kernel,
        mesh=mesh,
        out_type=[
            jax.ShapeDtypeStruct((B, ED), jnp.float32),
            jax.ShapeDtypeStruct((B, ED), jnp.float32),
        ],
        scratch_types=[
            pltpu.VMEM((CH,), jnp.int32),
            pltpu.VMEM((CH,), jnp.int32),
            pltpu.VMEM((CH, ED), jnp.float32),
            pltpu.VMEM((CH, ED), jnp.float32),
            pltpu.SemaphoreType.DMA,
        ],
    )
    def body(uid_h, mid_h, uemb_h, memb_h, ulat_h, mlat_h,
             uix, mix, ub, mb, sem):
        wid = lax.axis_index("s") * NC + lax.axis_index("c")
        base = wid * BPW
        for c in range(NCH):
            off = base + c * CH
            pltpu.sync_copy(uid_h.at[pl.ds(off, CH)], uix)
            pltpu.sync_copy(mid_h.at[pl.ds(off, CH)], mix)
            cu = pltpu.async_copy(uemb_h.at[uix], ub, sem)
            cm = pltpu.async_copy(memb_h.at[mix], mb, sem)
            cu.wait()
            cm.wait()
            pltpu.sync_copy(ub, ulat_h.at[pl.ds(off, CH)])
            pltpu.sync_copy(mb, mlat_h.at[pl.ds(off, CH)])

    return body(uid, mid, uemb, memb)


BLK = 1024


def _tc_body(u_ref, m_ref, ids_ref, wu_ref, ct_ref, wm_ref, b_ref, out_ref):
    g = ids_ref[0, 0, :]
    a = ids_ref[0, 1, :] + 2
    o = ids_ref[0, 2, :] + 9
    z = ids_ref[0, 3, :] + 30
    r = lax.broadcasted_iota(jnp.int32, (BLK, CT), 1)
    mh = ((g[:, None] == r).astype(jnp.float32)
          + (a[:, None] == r).astype(jnp.float32)
          + (o[:, None] == r).astype(jnp.float32)
          + (z[:, None] == r).astype(jnp.float32))
    meta32 = jnp.dot(mh, ct_ref[...], preferred_element_type=jnp.float32)
    t = jnp.dot(u_ref[...], wu_ref[...], preferred_element_type=jnp.float32)
    t += jnp.dot(meta32, wm_ref[...], preferred_element_type=jnp.float32)
    t += b_ref[...]
    out_ref[...] = jnp.sum(t * m_ref[...], axis=1)


def _tc_call(ulat, mlat, ids3, wu, ct, wm, bb):
    grid = (B // BLK,)
    row = lambda i: (i, 0)
    rep = lambda i: (0, 0)
    return pl.pallas_call(
        _tc_body,
        grid=grid,
        in_specs=[
            pl.BlockSpec((BLK, ED), row),
            pl.BlockSpec((BLK, ED), row),
            pl.BlockSpec((1, 4, BLK), lambda i: (i, 0, 0)),
            pl.BlockSpec((ED, ED), rep),
            pl.BlockSpec((CT, 4 * MD), rep),
            pl.BlockSpec((4 * MD, ED), rep),
            pl.BlockSpec((1, ED), rep),
        ],
        out_specs=pl.BlockSpec((BLK,), lambda i: (i,)),
        out_shape=jax.ShapeDtypeStruct((B,), jnp.float32),
    )(ulat, mlat, ids3, wu, ct, wm, bb)


def kernel(user_id, movie_id, gender, age, occupation, zip_code,
           user_emb, movie_emb, gender_emb, age_emb, occupation_emb, zip_emb,
           W, b):
    ct = jnp.zeros((CT, 4 * MD), jnp.float32)
    ct = ct.at[0:2, 0:MD].set(gender_emb)
    ct = ct.at[2:9, MD:2 * MD].set(age_emb)
    ct = ct.at[9:30, 2 * MD:3 * MD].set(occupation_emb)
    ct = ct.at[30:130, 3 * MD:4 * MD].set(zip_emb)
    wu = W[:ED]
    wm = W[ED:]
    bb = b.reshape(1, ED)
    ids3 = (jnp.stack([gender, age, occupation, zip_code])
            .reshape(4, B // BLK, BLK).transpose(1, 0, 2))

    ulat, mlat = _sc_gather(user_id, movie_id, user_emb, movie_emb)
    return _tc_call(ulat, mlat, ids3, wu, ct, wm, bb)
